# Initial kernel scaffold; baseline (speedup 1.0000x reference)
import jax
import jax.numpy as jnp
from jax.experimental import pallas as pl


def kernel(x, edge_index, Wn1, Ws1, bb1, Wn2, Ws2, bb2, Wn3, Ws3, bb3, g1, be1, rm1, rv1, g2, be2, rm2, rv2, Wc1, bc1, Wc2, bc2):
    raise NotImplementedError("write your pallas kernel here")


# speedup vs baseline: 3.5510x; 3.5510x over previous
import functools

import jax
import jax.numpy as jnp
from jax import lax
from jax.experimental import pallas as pl
from jax.experimental.pallas import tpu as pltpu
from jax.experimental.pallas import tpu_sc as plsc

_N = 10000
_E = 320000
_D = 128
_EPS = 1e-5

_NC = 2
_NS = 16
_NW = _NC * _NS
_C = 128
_CHUNKS = 79
_EPT = _C * _CHUNKS
_EP = _EPT * _NW
_NP = 10240
_RPT = _NP // _NS
_DUMMY = _N + 8

_mesh = plsc.VectorSubcoreMesh(core_axis_name="c", subcore_axis_name="s")

_sc_scratch = [
    pltpu.VMEM((_C,), jnp.int32),
    pltpu.VMEM((_C,), jnp.int32),
    pltpu.VMEM((_C, _D), jnp.float32),
    pltpu.VMEM_SHARED((_NP, _D), jnp.float32),
    pltpu.SemaphoreType.DMA,
]
_sc_out = jax.ShapeDtypeStruct((_NC, _NP, _D), jnp.float32)


def _fill(buf, val):
  @pl.loop(0, _C)
  def _(r):
    @pl.loop(0, _D // 16)
    def _(c16):
      buf[r, pl.ds(c16 * 16, 16)] = val


@functools.partial(pl.kernel, mesh=_mesh, out_type=_sc_out,
                   scratch_types=_sc_scratch)
def _segsum(h_hbm, src_hbm, dst_hbm, out_hbm, idx_s, idx_d, rows, acc_sh, sem):
  cid = lax.axis_index("c")
  sid = lax.axis_index("s")
  wid = sid * _NC + cid
  row0 = sid * _RPT

  _fill(rows, jnp.zeros((16,), jnp.float32))

  @pl.loop(0, _RPT, step=_C)
  def _(r0):
    pltpu.sync_copy(rows, acc_sh.at[pl.ds(row0 + r0, _C)])

  plsc.subcore_barrier()

  ebase = wid * _EPT

  @pl.loop(0, _CHUNKS)
  def _(j):
    b = ebase + j * _C
    pltpu.sync_copy(src_hbm.at[pl.ds(b, _C)], idx_s)
    pltpu.sync_copy(dst_hbm.at[pl.ds(b, _C)], idx_d)
    pltpu.async_copy(h_hbm.at[idx_s], rows, sem).wait()
    pltpu.sync_copy(rows, acc_sh.at[idx_d], add=True)

  plsc.subcore_barrier()

  @pl.loop(0, _RPT, step=_C)
  def _(r0):
    r = row0 + r0
    pltpu.sync_copy(acc_sh.at[pl.ds(r, _C)], out_hbm.at[cid, pl.ds(r, _C)])


@functools.partial(pl.kernel, mesh=_mesh, out_type=_sc_out,
                   scratch_types=_sc_scratch)
def _degree(dst_hbm, out_hbm, idx_s, idx_d, rows, acc_sh, sem):
  del idx_s, sem
  cid = lax.axis_index("c")
  sid = lax.axis_index("s")
  wid = sid * _NC + cid
  row0 = sid * _RPT

  _fill(rows, jnp.zeros((16,), jnp.float32))

  @pl.loop(0, _RPT, step=_C)
  def _(r0):
    pltpu.sync_copy(rows, acc_sh.at[pl.ds(row0 + r0, _C)])

  _fill(rows, jnp.ones((16,), jnp.float32))

  plsc.subcore_barrier()

  ebase = wid * _EPT

  @pl.loop(0, _CHUNKS)
  def _(j):
    pltpu.sync_copy(dst_hbm.at[pl.ds(ebase + j * _C, _C)], idx_d)
    pltpu.sync_copy(rows, acc_sh.at[idx_d], add=True)

  plsc.subcore_barrier()

  @pl.loop(0, _RPT, step=_C)
  def _(r0):
    r = row0 + r0
    pltpu.sync_copy(acc_sh.at[pl.ds(r, _C)], out_hbm.at[cid, pl.ds(r, _C)])


_BLK = 512
_GRID = _NP // _BLK


def _mean_from_parts(p_ref, d_ref):
  deg = (d_ref[0] + d_ref[1])[:, 0:1]
  rdeg = 1.0 / jnp.maximum(deg, 1.0)
  return (p_ref[0] + p_ref[1]) * rdeg


def _layer_body(p_ref, d_ref, h_ref, wn_ref, ws_ref, b_ref, s_ref, t_ref, o_ref):
  mean = _mean_from_parts(p_ref, d_ref)
  z = (jnp.dot(mean, wn_ref[...], preferred_element_type=jnp.float32)
       + jnp.dot(h_ref[...], ws_ref[...], preferred_element_type=jnp.float32)
       + b_ref[...])
  o_ref[...] = jnp.maximum(z * s_ref[...] + t_ref[...], 0.0)


def _final_body(p_ref, d_ref, h_ref, wn_ref, ws_ref, b_ref,
                wc1_ref, bc1_ref, wc2_ref, bc2_ref, o_ref):
  mean = _mean_from_parts(p_ref, d_ref)
  z = (jnp.dot(mean, wn_ref[...], preferred_element_type=jnp.float32)
       + jnp.dot(h_ref[...], ws_ref[...], preferred_element_type=jnp.float32)
       + b_ref[...])
  c1 = jnp.maximum(
      jnp.dot(z, wc1_ref[...], preferred_element_type=jnp.float32) + bc1_ref[...],
      0.0)
  o_ref[...] = (jnp.dot(c1, wc2_ref[...], preferred_element_type=jnp.float32)
                + bc2_ref[...])


_p_spec = pl.BlockSpec((2, _BLK, _D), lambda i: (0, i, 0))
_h_spec = pl.BlockSpec((_BLK, _D), lambda i: (i, 0))
_w_spec = pl.BlockSpec((_D, _D), lambda i: (0, 0))
_v_spec = pl.BlockSpec((1, _D), lambda i: (0, 0))

_layer_call = pl.pallas_call(
    _layer_body,
    grid=(_GRID,),
    in_specs=[_p_spec, _p_spec, _h_spec, _w_spec, _w_spec,
              _v_spec, _v_spec, _v_spec],
    out_specs=_h_spec,
    out_shape=jax.ShapeDtypeStruct((_NP, _D), jnp.float32),
)

_final_call = pl.pallas_call(
    _final_body,
    grid=(_GRID,),
    in_specs=[_p_spec, _p_spec, _h_spec, _w_spec, _w_spec, _v_spec,
              _w_spec, _v_spec, _w_spec, _v_spec],
    out_specs=_h_spec,
    out_shape=jax.ShapeDtypeStruct((_NP, _D), jnp.float32),
)


def kernel(x, edge_index, Wn1, Ws1, bb1, Wn2, Ws2, bb2, Wn3, Ws3, bb3,
           g1, be1, rm1, rv1, g2, be2, rm2, rv2, Wc1, bc1, Wc2, bc2):
  f32 = jnp.float32
  x_p = jnp.zeros((_NP, _D), f32).at[:_N].set(x)
  src = jnp.concatenate([edge_index[0], jnp.zeros((_EP - _E,), jnp.int32)])
  dst = jnp.concatenate([edge_index[1], jnp.full((_EP - _E,), _DUMMY, jnp.int32)])

  s1 = g1 / jnp.sqrt(rv1 + _EPS)
  t1 = be1 - rm1 * s1
  s2 = g2 / jnp.sqrt(rv2 + _EPS)
  t2 = be2 - rm2 * s2

  row = lambda v: v.reshape(1, _D)

  hh = Wc1.shape[1]
  ss = Wc2.shape[1]
  wc1p = jnp.zeros((_D, _D), f32).at[:, :hh].set(Wc1)
  bc1p = jnp.zeros((_D,), f32).at[:hh].set(bc1)
  wc2p = jnp.zeros((_D, _D), f32).at[:hh, :ss].set(Wc2)
  bc2p = jnp.zeros((_D,), f32).at[:ss].set(bc2)

  degp = _degree(dst)
  agg1 = _segsum(x_p, src, dst)
  h1 = _layer_call(agg1, degp, x_p, Wn1, Ws1, row(bb1), row(s1), row(t1))
  agg2 = _segsum(h1, src, dst)
  h2 = _layer_call(agg2, degp, h1, Wn2, Ws2, row(bb2), row(s2), row(t2))
  agg3 = _segsum(h2, src, dst)
  out = _final_call(agg3, degp, h2, Wn3, Ws3, row(bb3),
                    wc1p, row(bc1p), wc2p, row(bc2p))
  return out[:_N, :3]
